# baseline (device time: 50039 ns/iter reference)
import jax
import jax.numpy as jnp
from jax import lax
from jax.experimental import pallas as pl
from jax.experimental.pallas import tpu as pltpu

NZ = 4
NP = 8
CHUNK = 256
F = 4096
W = F // NP
K = 4
WS = W // K


def _ring_xy(r):
    rx = jnp.where(r < 4, 0, 1)
    ry = jnp.where(r < 4, r, 7 - r)
    return rx, ry


def kernel(x, dy):
    my_x = lax.axis_index("x")
    my_y = lax.axis_index("y")
    my_r = jnp.where(my_x == 0, my_y, 7 - my_y)

    dy_slice = lax.dynamic_slice(dy, (0, my_r * W), (dy.shape[0], W))
    xt = x.T.reshape(NZ, CHUNK, x.shape[0])

    def body(xt_ref, dys_ref, out_ref, comm, ag,
             rs_send, rs_recv, cw_send, cw_recv, ccw_send, ccw_recv):
        my_x = lax.axis_index("x")
        my_y = lax.axis_index("y")
        my_z = lax.axis_index("z")
        my_r = jnp.where(my_x == 0, my_y, 7 - my_y)

        z_left = (my_z + NZ - 1) % NZ
        z_right = (my_z + 1) % NZ
        rl_x, rl_y = _ring_xy((my_r + NP - 1) % NP)
        rr_x, rr_y = _ring_xy((my_r + 1) % NP)

        def n_cw(k):
            return 4 if k % 2 == 0 else 3

        def n_ccw(k):
            return 3 if k % 2 == 0 else 4

        def rs_rdma(k, s):
            return pltpu.make_async_remote_copy(
                src_ref=comm.at[k, s],
                dst_ref=comm.at[k, s + 1],
                send_sem=rs_send.at[k, s],
                recv_sem=rs_recv.at[k, s],
                device_id=(my_x, my_y, z_right),
                device_id_type=pl.DeviceIdType.MESH,
            )

        def ag_rdma(k, direction, h):
            if direction == "cw":
                o = (my_r + NP - h) % NP
                send, recv = cw_send, cw_recv
                dev = (rr_x, rr_y, my_z)
            else:
                o = (my_r + h) % NP
                send, recv = ccw_send, ccw_recv
                dev = (rl_x, rl_y, my_z)
            return pltpu.make_async_remote_copy(
                src_ref=ag.at[k, o],
                dst_ref=ag.at[k, o],
                send_sem=send.at[k, h],
                recv_sem=recv.at[k, h],
                device_id=dev,
                device_id_type=pl.DeviceIdType.MESH,
            )

        def ag_start(k, h):
            started = []
            if h < n_cw(k):
                r = ag_rdma(k, "cw", h)
                r.start()
                started.append(r)
            if h < n_ccw(k):
                r = ag_rdma(k, "ccw", h)
                r.start()
                started.append(r)
            return started

        barrier_sem = pltpu.get_barrier_semaphore()
        for dev in (
            (my_x, my_y, z_left),
            (my_x, my_y, z_right),
            (rl_x, rl_y, my_z),
            (rr_x, rr_y, my_z),
        ):
            pl.semaphore_signal(
                barrier_sem, inc=1, device_id=dev,
                device_id_type=pl.DeviceIdType.MESH,
            )
        pl.semaphore_wait(barrier_sem, 4)

        j0 = (my_z + NZ - 1) % NZ

        def contrib(j, k):
            return lax.dot_general(
                xt_ref[j],
                dys_ref[:, k * WS:(k + 1) * WS],
                (((1,), (0,)), ((), ())),
                preferred_element_type=jnp.float32,
            )

        rs_live = {}
        for k in range(K):
            comm[k, 0] = contrib(j0, k)
            r = rs_rdma(k, 0)
            r.start()
            rs_live[(k, 0)] = r

        ag_live = {}
        for s in range(NZ - 1):
            for k in range(K):
                rs_live[(k, s)].wait()
                j = (my_z + 2 * NZ - 2 - s) % NZ
                if s < NZ - 2:
                    comm[k, s + 1] = comm[k, s + 1] + contrib(j, k)
                    r = rs_rdma(k, s + 1)
                    r.start()
                    rs_live[(k, s + 1)] = r
                else:
                    ag[k, pl.ds(my_r, 1)] = (
                        comm[k, s + 1] + contrib(j, k)
                    )[jnp.newaxis]
                    ag_live[(k, 0)] = ag_start(k, 0)

        for h in range(4):
            for k in range(K):
                for r in ag_live[(k, h)]:
                    r.wait()
                if h < 3:
                    ag_live[(k, h + 1)] = ag_start(k, h + 1)
                else:
                    for j in range(NP):
                        out_ref[:, j * W + k * WS:j * W + (k + 1) * WS] = ag[k, j]

    return pl.pallas_call(
        body,
        out_shape=jax.ShapeDtypeStruct((CHUNK, F), jnp.float32),
        in_specs=[
            pl.BlockSpec(memory_space=pltpu.VMEM),
            pl.BlockSpec(memory_space=pltpu.VMEM),
        ],
        out_specs=pl.BlockSpec(memory_space=pltpu.VMEM),
        scratch_shapes=[
            pltpu.VMEM((K, NZ, CHUNK, WS), jnp.float32),
            pltpu.VMEM((K, NP, CHUNK, WS), jnp.float32),
            pltpu.SemaphoreType.DMA((K, NZ - 1)),
            pltpu.SemaphoreType.DMA((K, NZ - 1)),
            pltpu.SemaphoreType.DMA((K, 4)),
            pltpu.SemaphoreType.DMA((K, 4)),
            pltpu.SemaphoreType.DMA((K, 4)),
            pltpu.SemaphoreType.DMA((K, 4)),
        ],
        compiler_params=pltpu.CompilerParams(collective_id=0),
    )(xt, dy_slice)


# device time: 49776 ns/iter; 1.0053x vs baseline; 1.0053x over previous
import jax
import jax.numpy as jnp
from jax import lax
from jax.experimental import pallas as pl
from jax.experimental.pallas import tpu as pltpu

NZ = 4
NP = 8
CHUNK = 256
F = 4096
W = F // NP
K = 4
WS = W // K


def _ring_xy(r):
    rx = jnp.where(r < 4, 0, 1)
    ry = jnp.where(r < 4, r, 7 - r)
    return rx, ry


def kernel(x, dy):
    my_x = lax.axis_index("x")
    my_y = lax.axis_index("y")
    my_r = jnp.where(my_x == 0, my_y, 7 - my_y)

    dy_slice = lax.dynamic_slice(dy, (0, my_r * W), (dy.shape[0], W))
    xt = x.T.reshape(NZ, CHUNK, x.shape[0])

    def body(xt_ref, dys_ref, out_ref, comm, ag, pc,
             rs_send, rs_recv, cw_send, cw_recv, ccw_send, ccw_recv):
        my_x = lax.axis_index("x")
        my_y = lax.axis_index("y")
        my_z = lax.axis_index("z")
        my_r = jnp.where(my_x == 0, my_y, 7 - my_y)

        z_left = (my_z + NZ - 1) % NZ
        z_right = (my_z + 1) % NZ
        rl_x, rl_y = _ring_xy((my_r + NP - 1) % NP)
        rr_x, rr_y = _ring_xy((my_r + 1) % NP)

        def n_cw(k):
            return 4 if k % 2 == 0 else 3

        def n_ccw(k):
            return 3 if k % 2 == 0 else 4

        def rs_rdma(k, s):
            return pltpu.make_async_remote_copy(
                src_ref=comm.at[k, s],
                dst_ref=comm.at[k, s + 1],
                send_sem=rs_send.at[k, s],
                recv_sem=rs_recv.at[k, s],
                device_id=(my_x, my_y, z_right),
                device_id_type=pl.DeviceIdType.MESH,
            )

        def ag_rdma(k, direction, h):
            if direction == "cw":
                o = (my_r + NP - h) % NP
                send, recv = cw_send, cw_recv
                dev = (rr_x, rr_y, my_z)
            else:
                o = (my_r + h) % NP
                send, recv = ccw_send, ccw_recv
                dev = (rl_x, rl_y, my_z)
            return pltpu.make_async_remote_copy(
                src_ref=ag.at[k, o],
                dst_ref=ag.at[k, o],
                send_sem=send.at[k, h],
                recv_sem=recv.at[k, h],
                device_id=dev,
                device_id_type=pl.DeviceIdType.MESH,
            )

        def ag_start(k, h):
            started = []
            if h < n_cw(k):
                r = ag_rdma(k, "cw", h)
                r.start()
                started.append(r)
            if h < n_ccw(k):
                r = ag_rdma(k, "ccw", h)
                r.start()
                started.append(r)
            return started

        barrier_sem = pltpu.get_barrier_semaphore()
        for dev in (
            (my_x, my_y, z_left),
            (my_x, my_y, z_right),
            (rl_x, rl_y, my_z),
            (rr_x, rr_y, my_z),
        ):
            pl.semaphore_signal(
                barrier_sem, inc=1, device_id=dev,
                device_id_type=pl.DeviceIdType.MESH,
            )
        pl.semaphore_wait(barrier_sem, 4)

        j0 = (my_z + NZ - 1) % NZ

        def contrib(j, k):
            return lax.dot_general(
                xt_ref[j],
                dys_ref[:, k * WS:(k + 1) * WS],
                (((1,), (0,)), ((), ())),
                preferred_element_type=jnp.float32,
            )

        rs_live = {}
        for k in range(K):
            comm[k, 0] = contrib(j0, k)
            r = rs_rdma(k, 0)
            r.start()
            rs_live[(k, 0)] = r

        ag_live = {}
        for s in range(NZ - 1):
            j = (my_z + 2 * NZ - 2 - s) % NZ
            for k in range(K):
                pc[k] = contrib(j, k)
            for k in range(K):
                rs_live[(k, s)].wait()
                if s < NZ - 2:
                    comm[k, s + 1] = comm[k, s + 1] + pc[k]
                    r = rs_rdma(k, s + 1)
                    r.start()
                    rs_live[(k, s + 1)] = r
                else:
                    ag[k, pl.ds(my_r, 1)] = (comm[k, s + 1] + pc[k])[jnp.newaxis]
                    ag_live[(k, 0)] = ag_start(k, 0)

        for h in range(4):
            for k in range(K):
                for r in ag_live[(k, h)]:
                    r.wait()
                if h < 3:
                    ag_live[(k, h + 1)] = ag_start(k, h + 1)
                else:
                    for j in range(NP):
                        out_ref[:, j * W + k * WS:j * W + (k + 1) * WS] = ag[k, j]

    return pl.pallas_call(
        body,
        out_shape=jax.ShapeDtypeStruct((CHUNK, F), jnp.float32),
        in_specs=[
            pl.BlockSpec(memory_space=pltpu.VMEM),
            pl.BlockSpec(memory_space=pltpu.VMEM),
        ],
        out_specs=pl.BlockSpec(memory_space=pltpu.VMEM),
        scratch_shapes=[
            pltpu.VMEM((K, NZ, CHUNK, WS), jnp.float32),
            pltpu.VMEM((K, NP, CHUNK, WS), jnp.float32),
            pltpu.VMEM((K, CHUNK, WS), jnp.float32),
            pltpu.SemaphoreType.DMA((K, NZ - 1)),
            pltpu.SemaphoreType.DMA((K, NZ - 1)),
            pltpu.SemaphoreType.DMA((K, 4)),
            pltpu.SemaphoreType.DMA((K, 4)),
            pltpu.SemaphoreType.DMA((K, 4)),
            pltpu.SemaphoreType.DMA((K, 4)),
        ],
        compiler_params=pltpu.CompilerParams(collective_id=0),
    )(xt, dy_slice)


# device time: 34880 ns/iter; 1.4346x vs baseline; 1.4271x over previous
import jax
import jax.numpy as jnp
from jax import lax
from jax.experimental import pallas as pl
from jax.experimental.pallas import tpu as pltpu

import os
PROBE_RS_ONLY = os.environ.get("PROBE_RS_ONLY") == "1"
PROBE_AG_ONLY = os.environ.get("PROBE_AG_ONLY") == "1"

NZ = 4
NP = 8
CHUNK = 256
F = 4096
W = F // NP
K = 4
WS = W // K


def _ring_xy(r):
    rx = jnp.where(r < 4, 0, 1)
    ry = jnp.where(r < 4, r, 7 - r)
    return rx, ry


def kernel(x, dy):
    my_x = lax.axis_index("x")
    my_y = lax.axis_index("y")
    my_r = jnp.where(my_x == 0, my_y, 7 - my_y)

    dy_slice = lax.dynamic_slice(dy, (0, my_r * W), (dy.shape[0], W))
    xt = x.T.reshape(NZ, CHUNK, x.shape[0])

    def body(xt_ref, dys_ref, out_ref, comm, ag, pc,
             rs_send, rs_recv, cw_send, cw_recv, ccw_send, ccw_recv):
        my_x = lax.axis_index("x")
        my_y = lax.axis_index("y")
        my_z = lax.axis_index("z")
        my_r = jnp.where(my_x == 0, my_y, 7 - my_y)

        z_left = (my_z + NZ - 1) % NZ
        z_right = (my_z + 1) % NZ
        rl_x, rl_y = _ring_xy((my_r + NP - 1) % NP)
        rr_x, rr_y = _ring_xy((my_r + 1) % NP)

        def n_cw(k):
            return 4 if k % 2 == 0 else 3

        def n_ccw(k):
            return 3 if k % 2 == 0 else 4

        def rs_rdma(k, s):
            return pltpu.make_async_remote_copy(
                src_ref=comm.at[k, s],
                dst_ref=comm.at[k, s + 1],
                send_sem=rs_send.at[k, s],
                recv_sem=rs_recv.at[k, s],
                device_id=(my_x, my_y, z_right),
                device_id_type=pl.DeviceIdType.MESH,
            )

        def ag_rdma(k, direction, h):
            if direction == "cw":
                o = (my_r + NP - h) % NP
                send, recv = cw_send, cw_recv
                dev = (rr_x, rr_y, my_z)
            else:
                o = (my_r + h) % NP
                send, recv = ccw_send, ccw_recv
                dev = (rl_x, rl_y, my_z)
            return pltpu.make_async_remote_copy(
                src_ref=ag.at[k, o],
                dst_ref=ag.at[k, o],
                send_sem=send.at[k, h],
                recv_sem=recv.at[k, h],
                device_id=dev,
                device_id_type=pl.DeviceIdType.MESH,
            )

        def ag_start(k, h):
            started = []
            if h < n_cw(k):
                r = ag_rdma(k, "cw", h)
                r.start()
                started.append(r)
            if h < n_ccw(k):
                r = ag_rdma(k, "ccw", h)
                r.start()
                started.append(r)
            return started

        barrier_sem = pltpu.get_barrier_semaphore()
        for dev in (
            (my_x, my_y, z_left),
            (my_x, my_y, z_right),
            (rl_x, rl_y, my_z),
            (rr_x, rr_y, my_z),
        ):
            pl.semaphore_signal(
                barrier_sem, inc=1, device_id=dev,
                device_id_type=pl.DeviceIdType.MESH,
            )
        pl.semaphore_wait(barrier_sem, 4)

        j0 = (my_z + NZ - 1) % NZ

        def contrib(j, k):
            return lax.dot_general(
                xt_ref[j],
                dys_ref[:, k * WS:(k + 1) * WS],
                (((1,), (0,)), ((), ())),
                preferred_element_type=jnp.float32,
            )

        ag_live = {}
        if PROBE_AG_ONLY:
            for k in range(K):
                ag[k, pl.ds(my_r, 1)] = contrib(j0, k)[jnp.newaxis]
                ag_live[(k, 0)] = ag_start(k, 0)

        rs_live = {}
        for k in range(K):
            if PROBE_AG_ONLY:
                break
            comm[k, 0] = contrib(j0, k)
            r = rs_rdma(k, 0)
            r.start()
            rs_live[(k, 0)] = r

        for s in range(0 if not PROBE_AG_ONLY else 99, NZ - 1):
            j = (my_z + 2 * NZ - 2 - s) % NZ
            for k in range(K):
                pc[k] = contrib(j, k)
            for k in range(K):
                rs_live[(k, s)].wait()
                if s < NZ - 2:
                    comm[k, s + 1] = comm[k, s + 1] + pc[k]
                    r = rs_rdma(k, s + 1)
                    r.start()
                    rs_live[(k, s + 1)] = r
                else:
                    ag[k, pl.ds(my_r, 1)] = (comm[k, s + 1] + pc[k])[jnp.newaxis]
                    if not PROBE_RS_ONLY:
                        ag_live[(k, 0)] = ag_start(k, 0)

        if PROBE_RS_ONLY:
            for k in range(K):
                for j in range(NP):
                    out_ref[:, j * W + k * WS:j * W + (k + 1) * WS] = ag[k, j]
        else:
            for h in range(4):
                for k in range(K):
                    for r in ag_live[(k, h)]:
                        r.wait()
                    if h < 3:
                        ag_live[(k, h + 1)] = ag_start(k, h + 1)
                    else:
                        for j in range(NP):
                            out_ref[:, j * W + k * WS:j * W + (k + 1) * WS] = ag[k, j]

    return pl.pallas_call(
        body,
        out_shape=jax.ShapeDtypeStruct((CHUNK, F), jnp.float32),
        in_specs=[
            pl.BlockSpec(memory_space=pltpu.VMEM),
            pl.BlockSpec(memory_space=pltpu.VMEM),
        ],
        out_specs=pl.BlockSpec(memory_space=pltpu.VMEM),
        scratch_shapes=[
            pltpu.VMEM((K, NZ, CHUNK, WS), jnp.float32),
            pltpu.VMEM((K, NP, CHUNK, WS), jnp.float32),
            pltpu.VMEM((K, CHUNK, WS), jnp.float32),
            pltpu.SemaphoreType.DMA((K, NZ - 1)),
            pltpu.SemaphoreType.DMA((K, NZ - 1)),
            pltpu.SemaphoreType.DMA((K, 4)),
            pltpu.SemaphoreType.DMA((K, 4)),
            pltpu.SemaphoreType.DMA((K, 4)),
            pltpu.SemaphoreType.DMA((K, 4)),
        ],
        compiler_params=pltpu.CompilerParams(collective_id=0),
    )(xt, dy_slice)
